# baseline (speedup 1.0000x reference)
import functools

import jax
import jax.numpy as jnp
from jax import lax
from jax.experimental import pallas as pl
from jax.experimental.pallas import tpu as pltpu
from jax.experimental.pallas import tpu_sc as plsc

NUM_EMBEDDINGS = 1000000
EMBED_DIM = 64
BATCH = 16384

_INFO = plsc.get_sparse_core_info()
_NC, _NS, _L = _INFO.num_cores, _INFO.num_subcores, _INFO.num_lanes
_NW = _NC * _NS
_BPW = BATCH // _NW
_CHUNK = 128
_NCHUNK = _BPW // _CHUNK
_WIDE = 2 * EMBED_DIM


def _sc_kernel(nodes_hbm, table2_hbm, out_hbm, idx_v, gidx_v, off_v,
               rows_v, outt_v, sem_g, sem_o):
    wid = lax.axis_index("s") * _NC + lax.axis_index("c")
    base = wid * _BPW

    pltpu.sync_copy(nodes_hbm.at[pl.ds(base, _BPW)], idx_v)

    gathers = []
    for q in range(_NCHUNK):
        for jb in range(_CHUNK // _L):
            v = idx_v[pl.ds(q * _CHUNK + jb * _L, _L)]
            gidx_v[pl.ds(q * _CHUNK + jb * _L, _L)] = v >> 1
            off_v[pl.ds(q * _CHUNK + jb * _L, _L)] = (v & 1) << 6
        gathers.append(
            pltpu.async_copy(
                table2_hbm.at[gidx_v.at[pl.ds(q * _CHUNK, _CHUNK)]],
                rows_v.at[q],
                sem_g.at[q],
            )
        )

    iota = lax.iota(jnp.int32, _L)
    out_copies = []
    for q in range(_NCHUNK):
        gathers[q].wait()
        rows_q = rows_v.at[q]
        outt_q = outt_v.at[q]

        @plsc.parallel_loop(0, _CHUNK // _L, unroll=2)
        def body(jb, rows_q=rows_q, outt_q=outt_q, q=q):
            offv = off_v[pl.ds(q * _CHUNK + jb * _L, _L)]
            rowsel = jb * _L + iota
            for c in range(_L):
                dperm = (iota + c) & (_L - 1)
                for db in range(EMBED_DIM // _L):
                    dvec = dperm + (db * _L)
                    v = plsc.load_gather(rows_q, [rowsel, offv + dvec])
                    plsc.store_scatter(outt_q, [dvec, rowsel], v)

        out_copies.append(
            pltpu.async_copy(
                outt_q,
                out_hbm.at[:, pl.ds(base + q * _CHUNK, _CHUNK)],
                sem_o,
            )
        )
    for c in out_copies:
        c.wait()


@jax.jit
def _lookup_t(nodes, table):
    nodes1d = nodes.astype(jnp.int32)
    table2 = table.reshape(NUM_EMBEDDINGS // 2, _WIDE)
    mesh = plsc.VectorSubcoreMesh(core_axis_name="c", subcore_axis_name="s")
    f = functools.partial(
        pl.kernel,
        mesh=mesh,
        out_type=jax.ShapeDtypeStruct((EMBED_DIM, BATCH), jnp.float32),
        scratch_types=[
            pltpu.VMEM((_BPW,), jnp.int32),
            pltpu.VMEM((_BPW,), jnp.int32),
            pltpu.VMEM((_BPW,), jnp.int32),
            pltpu.VMEM((_NCHUNK, _CHUNK, _WIDE), jnp.float32),
            pltpu.VMEM((_NCHUNK, EMBED_DIM, _CHUNK), jnp.float32),
            pltpu.SemaphoreType.DMA((_NCHUNK,)),
            pltpu.SemaphoreType.DMA,
        ],
        compiler_params=pltpu.CompilerParams(
            needs_layout_passes=False, use_tc_tiling_on_sc=True
        ),
    )(_sc_kernel)
    return f(nodes1d, table2)


def kernel(nodes, table):
    return _lookup_t(nodes, table)


# speedup vs baseline: 1.6967x; 1.6967x over previous
import functools

import jax
import jax.numpy as jnp
from jax import lax
from jax.experimental import pallas as pl
from jax.experimental.pallas import tpu as pltpu
from jax.experimental.pallas import tpu_sc as plsc

NUM_EMBEDDINGS = 1000000
EMBED_DIM = 64
BATCH = 16384

_INFO = plsc.get_sparse_core_info()
_NC, _NS, _L = _INFO.num_cores, _INFO.num_subcores, _INFO.num_lanes
_NW = _NC * _NS
_BPW = BATCH // _NW
_CHUNK = 128
_NCHUNK = _BPW // _CHUNK


def _sc_kernel(nodes_hbm, table_hbm, out_hbm, idx_v, rows_v, outt_v,
               sem_g, sem_o):
    wid = lax.axis_index("s") * _NC + lax.axis_index("c")
    base = wid * _BPW

    pltpu.sync_copy(nodes_hbm.at[pl.ds(base, _BPW)], idx_v)

    for q in range(_NCHUNK):
        def issue(jb, carry, q=q):
            v16 = idx_v[pl.ds(q * _CHUNK + jb * _L, _L)]
            for k in range(_L):
                row = v16[k]
                pltpu.async_copy(
                    table_hbm.at[row],
                    rows_v.at[q, jb * _L + k],
                    sem_g.at[q],
                )
            return carry

        lax.fori_loop(0, _CHUNK // _L, issue, 0)

    iota = lax.iota(jnp.int32, _L)
    out_copies = []
    for q in range(_NCHUNK):
        pltpu.make_async_copy(
            table_hbm.at[pl.ds(0, _CHUNK)], rows_v.at[q], sem_g.at[q]
        ).wait()
        rows_q = rows_v.at[q]
        outt_q = outt_v.at[q]

        @plsc.parallel_loop(0, _CHUNK // _L, unroll=2)
        def body(jb, rows_q=rows_q, outt_q=outt_q):
            rowsel = jb * _L + iota
            for c in range(_L):
                dperm = (iota + c) & (_L - 1)
                for db in range(EMBED_DIM // _L):
                    dvec = dperm + (db * _L)
                    v = plsc.load_gather(rows_q, [rowsel, dvec])
                    plsc.store_scatter(outt_q, [dvec, rowsel], v)

        out_copies.append(
            pltpu.async_copy(
                outt_q,
                out_hbm.at[:, pl.ds(base + q * _CHUNK, _CHUNK)],
                sem_o,
            )
        )
    for c in out_copies:
        c.wait()


@jax.jit
def _lookup_t(nodes, table):
    nodes1d = nodes.astype(jnp.int32)
    mesh = plsc.VectorSubcoreMesh(core_axis_name="c", subcore_axis_name="s")
    f = functools.partial(
        pl.kernel,
        mesh=mesh,
        out_type=jax.ShapeDtypeStruct((EMBED_DIM, BATCH), jnp.float32),
        scratch_types=[
            pltpu.VMEM((_BPW,), jnp.int32),
            pltpu.VMEM((_NCHUNK, _CHUNK, EMBED_DIM), jnp.float32),
            pltpu.VMEM((_NCHUNK, EMBED_DIM, _CHUNK), jnp.float32),
            pltpu.SemaphoreType.DMA((_NCHUNK,)),
            pltpu.SemaphoreType.DMA,
        ],
        compiler_params=pltpu.CompilerParams(
            needs_layout_passes=False, use_tc_tiling_on_sc=True
        ),
    )(_sc_kernel)
    return f(nodes1d, table)


def kernel(nodes, table):
    return _lookup_t(nodes, table)
